# baseline (speedup 1.0000x reference)
import functools

import jax
import jax.numpy as jnp
from jax import lax
from jax.experimental import pallas as pl
from jax.experimental.pallas import tpu as pltpu
import jax.experimental.pallas.tpu_sc as plsc

N_NODES_R = 10000
NFEAT = 128
HW = NFEAT // 2
N_EDGES_R = 320000

NC = 2
NS = 16
NW = NC * NS
CHUNK = 128
CPW = 80
E_PAD = NW * CPW * CHUNK
N_PAD = 10240
RPT = N_PAD // NS

_mesh = lambda: plsc.VectorSubcoreMesh(core_axis_name="c", subcore_axis_name="s")


@functools.partial(
    pl.kernel,
    out_type=jax.ShapeDtypeStruct((NC, N_PAD), jnp.float32),
    mesh=_mesh(),
    scratch_types=[
        pltpu.VMEM((CPW, CHUNK), jnp.int32),
        pltpu.VMEM((CHUNK,), jnp.float32),
        pltpu.VMEM((RPT,), jnp.float32),
        pltpu.VMEM_SHARED((N_PAD,), jnp.float32),
    ],
)
def _deg_kernel(dst_hbm, out_hbm, dst_v, ones_v, zbuf, acc):
    c = lax.axis_index("c")
    s = lax.axis_index("s")
    wid = s * NC + c

    pltpu.sync_copy(dst_hbm.at[pl.ds(wid * CPW, CPW)], dst_v)

    one16 = jnp.ones((16,), jnp.float32)
    zero16 = jnp.zeros((16,), jnp.float32)
    for k in range(CHUNK // 16):
        ones_v[pl.ds(k * 16, 16)] = one16

    def zb(i, carry):
        zbuf[pl.ds(i * 16, 16)] = zero16
        return carry
    lax.fori_loop(0, RPT // 16, zb, 0)
    pltpu.sync_copy(zbuf, acc.at[pl.ds(s * RPT, RPT)])
    plsc.subcore_barrier()

    def body(j, carry):
        pltpu.sync_copy(ones_v, acc.at[dst_v.at[j]], add=True)
        return carry
    lax.fori_loop(0, CPW, body, 0)
    plsc.subcore_barrier()

    pltpu.sync_copy(acc.at[pl.ds(s * RPT, RPT)], out_hbm.at[c, pl.ds(s * RPT, RPT)])


_NBUF = 2
CPW0 = 160
CPW1 = 160 - CPW0
_STAGES0 = (56, 56, 48)
_STAGES1 = ()
_SMAX = 56


@functools.partial(
    pl.kernel,
    out_type=jax.ShapeDtypeStruct((NC, N_PAD, NFEAT), jnp.float32),
    mesh=_mesh(),
    scratch_types=[
        pltpu.VMEM((_SMAX, CHUNK), jnp.int32),
        pltpu.VMEM((_SMAX, CHUNK), jnp.int32),
        pltpu.VMEM((CHUNK, NFEAT), jnp.float32),
        pltpu.VMEM((CHUNK, NFEAT), jnp.float32),
        pltpu.VMEM_SHARED((N_PAD, NFEAT), jnp.float32),
        pltpu.SemaphoreType.DMA,
        pltpu.SemaphoreType.DMA,
    ],
)
def _agg_kernel(y_hbm, src_hbm, dst_hbm, out_hbm, src_v, dst_v, rows0, rows1,
                acc, gsem0, gsem1):
    rows = (rows0, rows1)
    gsem = (gsem0, gsem1)
    c = lax.axis_index("c")
    s = lax.axis_index("s")

    zero16 = jnp.zeros((16,), jnp.float32)
    def zrow(i, carry):
        for k in range(NFEAT // 16):
            rows[0][i, pl.ds(k * 16, 16)] = zero16
        return carry
    lax.fori_loop(0, CHUNK, zrow, 0)
    for t in range(RPT // CHUNK):
        pltpu.sync_copy(rows[0], acc.at[pl.ds(s * RPT + t * CHUNK, CHUNK)])
    plsc.subcore_barrier()

    def run_edges(row_base, stage_sizes):
        off = 0
        for sz in stage_sizes:
            pltpu.sync_copy(src_hbm.at[pl.ds(row_base + off, sz)],
                            src_v.at[pl.ds(0, sz)])
            pltpu.sync_copy(dst_hbm.at[pl.ds(row_base + off, sz)],
                            dst_v.at[pl.ds(0, sz)])
            for b in range(_NBUF):
                pltpu.async_copy(y_hbm.at[src_v.at[b]], rows[b], gsem[b])

            def body(q, carry):
                j0 = q * _NBUF
                for b in range(_NBUF):
                    pltpu.make_async_copy(y_hbm.at[src_v.at[j0 + b]], rows[b],
                                          gsem[b]).wait()
                    pltpu.sync_copy(rows[b], acc.at[dst_v.at[j0 + b]],
                                    add=True)
                    @pl.when(j0 + b + _NBUF < sz)
                    def _():
                        pltpu.async_copy(y_hbm.at[src_v.at[j0 + b + _NBUF]],
                                         rows[b], gsem[b])
                return carry
            lax.fori_loop(0, sz // _NBUF, body, 0)
            off += sz

    @pl.when(c == 0)
    def _():
        run_edges(s * CPW0, _STAGES0)

    @pl.when(c == 1)
    def _():
        run_edges(NS * CPW0 + s * CPW1, _STAGES1)

    plsc.subcore_barrier()

    pltpu.sync_copy(acc.at[pl.ds(s * RPT, RPT)],
                    out_hbm.at[c, pl.ds(s * RPT, RPT)])


_BLK = 640
_GRID = N_PAD // _BLK


def _row_mask(i):
    rid = i * _BLK + lax.broadcasted_iota(jnp.int32, (_BLK, 1), 0)
    return rid < N_NODES_R


def _tc1_body(x_ref, w1_ref, da_ref, db_ref, y1_ref, dinv_ref):
    i = pl.program_id(0)
    d = da_ref[...] + db_ref[...] + 1.0
    r = jnp.where(_row_mask(i), lax.rsqrt(d), 0.0)
    xw = jnp.dot(x_ref[...], w1_ref[...], preferred_element_type=jnp.float32)
    y1_ref[...] = r * xw
    dinv_ref[...] = r


def _tc2_body(y1_ref, aa_ref, ab_ref, dinv_ref, b1_ref, w2_ref, y2_ref):
    r = dinv_ref[...]
    a = aa_ref[...] + ab_ref[...] + y1_ref[...]
    h = jnp.maximum(r * a + b1_ref[...], 0.0)
    y2_ref[...] = r * jnp.dot(h, w2_ref[...], preferred_element_type=jnp.float32)


def _tc3_body(y2_ref, aa_ref, ab_ref, dinv_ref, b2_ref, wfc_ref, bfc_ref,
              h2_ref, lg_ref):
    r = dinv_ref[...]
    h2 = r * (aa_ref[...] + ab_ref[...] + y2_ref[...]) + b2_ref[...]
    h2_ref[...] = h2
    lg_ref[...] = jnp.dot(h2, wfc_ref[...], preferred_element_type=jnp.float32) + bfc_ref[...]


def _rows_spec(width=NFEAT):
    return pl.BlockSpec((_BLK, width), lambda i: (i, 0))


def _full_spec(shape):
    return pl.BlockSpec(shape, lambda i: tuple(0 for _ in shape))


def kernel(x, edge_index, W1, b1, W2, b2, Wfc, bfc):
    f32 = jnp.float32
    src = edge_index[0].astype(jnp.int32)
    dst = edge_index[1].astype(jnp.int32)
    pad_e = E_PAD - src.shape[0]
    fill = jnp.full((pad_e,), N_NODES_R, jnp.int32)
    src2 = jnp.concatenate([src, fill]).reshape(NW * CPW, CHUNK)
    dst2 = jnp.concatenate([dst, fill]).reshape(NW * CPW, CHUNK)

    x_pad = jnp.pad(x.astype(f32), ((0, N_PAD - N_NODES_R), (0, 0)))

    deg = _deg_kernel(dst2)
    degA = deg[0].reshape(N_PAD, 1)
    degB = deg[1].reshape(N_PAD, 1)

    y1, dinv = pl.pallas_call(
        _tc1_body,
        grid=(_GRID,),
        in_specs=[_rows_spec(), _full_spec((NFEAT, NFEAT)),
                  _rows_spec(1), _rows_spec(1)],
        out_specs=[_rows_spec(), _rows_spec(1)],
        out_shape=[jax.ShapeDtypeStruct((N_PAD, NFEAT), f32),
                   jax.ShapeDtypeStruct((N_PAD, 1), f32)],
    )(x_pad, W1.astype(f32), degA, degB)

    agg1 = _agg_kernel(y1, src2, dst2)

    y2 = pl.pallas_call(
        _tc2_body,
        grid=(_GRID,),
        in_specs=[_rows_spec(), _rows_spec(), _rows_spec(), _rows_spec(1),
                  _full_spec((1, NFEAT)), _full_spec((NFEAT, NFEAT))],
        out_specs=_rows_spec(),
        out_shape=jax.ShapeDtypeStruct((N_PAD, NFEAT), f32),
    )(y1, agg1[0], agg1[1], dinv, b1.astype(f32).reshape(1, NFEAT),
      W2.astype(f32))

    agg2 = _agg_kernel(y2, src2, dst2)

    ncls = Wfc.shape[1]
    wfc_pad = jnp.pad(Wfc.astype(f32), ((0, 0), (0, NFEAT - ncls)))
    bfc_pad = jnp.pad(bfc.astype(f32), (0, NFEAT - ncls)).reshape(1, NFEAT)

    h2, logits = pl.pallas_call(
        _tc3_body,
        grid=(_GRID,),
        in_specs=[_rows_spec(), _rows_spec(), _rows_spec(), _rows_spec(1),
                  _full_spec((1, NFEAT)), _full_spec((NFEAT, NFEAT)),
                  _full_spec((1, NFEAT))],
        out_specs=[_rows_spec(), _rows_spec()],
        out_shape=[jax.ShapeDtypeStruct((N_PAD, NFEAT), f32),
                   jax.ShapeDtypeStruct((N_PAD, NFEAT), f32)],
    )(y2, agg2[0], agg2[1], dinv, b2.astype(f32).reshape(1, NFEAT),
      wfc_pad, bfc_pad)

    return (h2[:N_NODES_R], logits[:N_NODES_R, :ncls])


# speedup vs baseline: 1.3312x; 1.3312x over previous
import functools

import jax
import jax.numpy as jnp
from jax import lax
from jax.experimental import pallas as pl
from jax.experimental.pallas import tpu as pltpu
import jax.experimental.pallas.tpu_sc as plsc

N_NODES_R = 10000
NFEAT = 128
HW = NFEAT // 2
N_EDGES_R = 320000

NC = 2
NS = 16
NW = NC * NS
CHUNK = 128
CPW = 80
E_PAD = NW * CPW * CHUNK
N_PAD = 10240
RPT = N_PAD // NS

_mesh = lambda: plsc.VectorSubcoreMesh(core_axis_name="c", subcore_axis_name="s")


@functools.partial(
    pl.kernel,
    out_type=jax.ShapeDtypeStruct((NC, N_PAD), jnp.float32),
    mesh=_mesh(),
    scratch_types=[
        pltpu.VMEM((CPW, CHUNK), jnp.int32),
        pltpu.VMEM((CHUNK,), jnp.float32),
        pltpu.VMEM((RPT,), jnp.float32),
        pltpu.VMEM_SHARED((N_PAD,), jnp.float32),
    ],
)
def _deg_kernel(dst_hbm, out_hbm, dst_v, ones_v, zbuf, acc):
    c = lax.axis_index("c")
    s = lax.axis_index("s")
    wid = s * NC + c

    pltpu.sync_copy(dst_hbm.at[pl.ds(wid * CPW, CPW)], dst_v)

    one16 = jnp.ones((16,), jnp.float32)
    zero16 = jnp.zeros((16,), jnp.float32)
    for k in range(CHUNK // 16):
        ones_v[pl.ds(k * 16, 16)] = one16

    def zb(i, carry):
        zbuf[pl.ds(i * 16, 16)] = zero16
        return carry
    lax.fori_loop(0, RPT // 16, zb, 0)
    pltpu.sync_copy(zbuf, acc.at[pl.ds(s * RPT, RPT)])
    plsc.subcore_barrier()

    def body(j, carry):
        pltpu.sync_copy(ones_v, acc.at[dst_v.at[j]], add=True)
        return carry
    lax.fori_loop(0, CPW, body, 0)
    plsc.subcore_barrier()

    pltpu.sync_copy(acc.at[pl.ds(s * RPT, RPT)], out_hbm.at[c, pl.ds(s * RPT, RPT)])


_NBUF = 2
CPW0 = 152
CPW1 = 160 - CPW0
_STAGES0 = (56, 56, 40)
_STAGES1 = (8,)
_SMAX = 56


@functools.partial(
    pl.kernel,
    out_type=jax.ShapeDtypeStruct((NC, N_PAD, NFEAT), jnp.float32),
    mesh=_mesh(),
    scratch_types=[
        pltpu.VMEM((_SMAX, CHUNK), jnp.int32),
        pltpu.VMEM((_SMAX, CHUNK), jnp.int32),
        pltpu.VMEM((CHUNK, NFEAT), jnp.float32),
        pltpu.VMEM((CHUNK, NFEAT), jnp.float32),
        pltpu.VMEM_SHARED((N_PAD, NFEAT), jnp.float32),
        pltpu.SemaphoreType.DMA,
        pltpu.SemaphoreType.DMA,
    ],
)
def _agg_kernel(y_hbm, src_hbm, dst_hbm, out_hbm, src_v, dst_v, rows0, rows1,
                acc, gsem0, gsem1):
    rows = (rows0, rows1)
    gsem = (gsem0, gsem1)
    c = lax.axis_index("c")
    s = lax.axis_index("s")

    zero16 = jnp.zeros((16,), jnp.float32)
    def zrow(i, carry):
        for k in range(NFEAT // 16):
            rows[0][i, pl.ds(k * 16, 16)] = zero16
        return carry
    lax.fori_loop(0, CHUNK, zrow, 0)
    for t in range(RPT // CHUNK):
        pltpu.sync_copy(rows[0], acc.at[pl.ds(s * RPT + t * CHUNK, CHUNK)])
    plsc.subcore_barrier()

    def run_edges(row_base, stage_sizes):
        off = 0
        for sz in stage_sizes:
            pltpu.sync_copy(src_hbm.at[pl.ds(row_base + off, sz)],
                            src_v.at[pl.ds(0, sz)])
            pltpu.sync_copy(dst_hbm.at[pl.ds(row_base + off, sz)],
                            dst_v.at[pl.ds(0, sz)])
            for b in range(_NBUF):
                pltpu.async_copy(y_hbm.at[src_v.at[b]], rows[b], gsem[b])

            def body(q, carry):
                j0 = q * _NBUF
                for b in range(_NBUF):
                    pltpu.make_async_copy(y_hbm.at[src_v.at[j0 + b]], rows[b],
                                          gsem[b]).wait()
                    pltpu.sync_copy(rows[b], acc.at[dst_v.at[j0 + b]],
                                    add=True)
                    @pl.when(j0 + b + _NBUF < sz)
                    def _():
                        pltpu.async_copy(y_hbm.at[src_v.at[j0 + b + _NBUF]],
                                         rows[b], gsem[b])
                return carry
            lax.fori_loop(0, sz // _NBUF, body, 0)
            off += sz

    @pl.when(c == 0)
    def _():
        run_edges(s * CPW0, _STAGES0)

    @pl.when(c == 1)
    def _():
        run_edges(NS * CPW0 + s * CPW1, _STAGES1)

    plsc.subcore_barrier()

    pltpu.sync_copy(acc.at[pl.ds(s * RPT, RPT)],
                    out_hbm.at[c, pl.ds(s * RPT, RPT)])


_BLK = 640
_GRID = N_PAD // _BLK


def _row_mask(i):
    rid = i * _BLK + lax.broadcasted_iota(jnp.int32, (_BLK, 1), 0)
    return rid < N_NODES_R


def _tc1_body(x_ref, w1_ref, da_ref, db_ref, y1_ref, dinv_ref):
    i = pl.program_id(0)
    d = da_ref[...] + db_ref[...] + 1.0
    r = jnp.where(_row_mask(i), lax.rsqrt(d), 0.0)
    xw = jnp.dot(x_ref[...], w1_ref[...], preferred_element_type=jnp.float32)
    y1_ref[...] = r * xw
    dinv_ref[...] = r


def _tc2_body(y1_ref, aa_ref, ab_ref, dinv_ref, b1_ref, w2_ref, y2_ref):
    r = dinv_ref[...]
    a = aa_ref[...] + ab_ref[...] + y1_ref[...]
    h = jnp.maximum(r * a + b1_ref[...], 0.0)
    y2_ref[...] = r * jnp.dot(h, w2_ref[...], preferred_element_type=jnp.float32)


def _tc3_body(y2_ref, aa_ref, ab_ref, dinv_ref, b2_ref, wfc_ref, bfc_ref,
              h2_ref, lg_ref):
    r = dinv_ref[...]
    h2 = r * (aa_ref[...] + ab_ref[...] + y2_ref[...]) + b2_ref[...]
    h2_ref[...] = h2
    lg_ref[...] = jnp.dot(h2, wfc_ref[...], preferred_element_type=jnp.float32) + bfc_ref[...]


def _rows_spec(width=NFEAT):
    return pl.BlockSpec((_BLK, width), lambda i: (i, 0))


def _full_spec(shape):
    return pl.BlockSpec(shape, lambda i: tuple(0 for _ in shape))


def kernel(x, edge_index, W1, b1, W2, b2, Wfc, bfc):
    f32 = jnp.float32
    src = edge_index[0].astype(jnp.int32)
    dst = edge_index[1].astype(jnp.int32)
    pad_e = E_PAD - src.shape[0]
    fill = jnp.full((pad_e,), N_NODES_R, jnp.int32)
    src2 = jnp.concatenate([src, fill]).reshape(NW * CPW, CHUNK)
    dst2 = jnp.concatenate([dst, fill]).reshape(NW * CPW, CHUNK)

    x_pad = jnp.pad(x.astype(f32), ((0, N_PAD - N_NODES_R), (0, 0)))

    deg = _deg_kernel(dst2)
    degA = deg[0].reshape(N_PAD, 1)
    degB = deg[1].reshape(N_PAD, 1)

    y1, dinv = pl.pallas_call(
        _tc1_body,
        grid=(_GRID,),
        in_specs=[_rows_spec(), _full_spec((NFEAT, NFEAT)),
                  _rows_spec(1), _rows_spec(1)],
        out_specs=[_rows_spec(), _rows_spec(1)],
        out_shape=[jax.ShapeDtypeStruct((N_PAD, NFEAT), f32),
                   jax.ShapeDtypeStruct((N_PAD, 1), f32)],
    )(x_pad, W1.astype(f32), degA, degB)

    agg1 = _agg_kernel(y1, src2, dst2)

    y2 = pl.pallas_call(
        _tc2_body,
        grid=(_GRID,),
        in_specs=[_rows_spec(), _rows_spec(), _rows_spec(), _rows_spec(1),
                  _full_spec((1, NFEAT)), _full_spec((NFEAT, NFEAT))],
        out_specs=_rows_spec(),
        out_shape=jax.ShapeDtypeStruct((N_PAD, NFEAT), f32),
    )(y1, agg1[0], agg1[1], dinv, b1.astype(f32).reshape(1, NFEAT),
      W2.astype(f32))

    agg2 = _agg_kernel(y2, src2, dst2)

    ncls = Wfc.shape[1]
    wfc_pad = jnp.pad(Wfc.astype(f32), ((0, 0), (0, NFEAT - ncls)))
    bfc_pad = jnp.pad(bfc.astype(f32), (0, NFEAT - ncls)).reshape(1, NFEAT)

    h2, logits = pl.pallas_call(
        _tc3_body,
        grid=(_GRID,),
        in_specs=[_rows_spec(), _rows_spec(), _rows_spec(), _rows_spec(1),
                  _full_spec((1, NFEAT)), _full_spec((NFEAT, NFEAT)),
                  _full_spec((1, NFEAT))],
        out_specs=[_rows_spec(), _rows_spec()],
        out_shape=[jax.ShapeDtypeStruct((N_PAD, NFEAT), f32),
                   jax.ShapeDtypeStruct((N_PAD, NFEAT), f32)],
    )(y2, agg2[0], agg2[1], dinv, b2.astype(f32).reshape(1, NFEAT),
      wfc_pad, bfc_pad)

    return (h2[:N_NODES_R], logits[:N_NODES_R, :ncls])
